# baseline (speedup 1.0000x reference)
import jax
import jax.numpy as jnp
from jax import lax
from jax.experimental import pallas as pl
from jax.experimental.pallas import tpu as pltpu
from jax.experimental.pallas import tpu_sc as plsc

N_NODES = 50000
N_CELLS = 100000
D = 128

NUM_CORES = 2
NUM_SUBCORES = 16
NUM_TILES = NUM_CORES * NUM_SUBCORES

STREAM_B = 128
CHUNK_STREAMS = 24
CHUNK_CELLS = CHUNK_STREAMS * STREAM_B // 3
NUM_CHUNKS = 125
NC_PAD = NUM_CHUNKS * CHUNK_CELLS
CHUNKS_PER_TILE = 4
LAST_REAL_CHUNK = (N_CELLS - 1) // CHUNK_CELLS
TRI_ROWS = NC_PAD * 3 // STREAM_B


def _q_matmul_body(u_ref, w_ref, q_ref):
    q_ref[...] = jnp.dot(u_ref[...], w_ref[...],
                         preferred_element_type=jnp.float32)


def _compute_q(u2d, wcat):
    blk = 5000
    return pl.pallas_call(
        _q_matmul_body,
        grid=(N_NODES // blk,),
        in_specs=[
            pl.BlockSpec((blk, D), lambda i: (i, 0)),
            pl.BlockSpec((D, 16), lambda i: (0, 0)),
        ],
        out_specs=pl.BlockSpec((blk, 16), lambda i: (i, 0)),
        out_shape=jax.ShapeDtypeStruct((N_NODES, 16), jnp.float32),
    )(u2d, wcat)


def _prep_body(cc_ref, vp_ref, ffd_ref, tri_ref, wcc_ref, wvp_ref, c_ref,
               bf_ref, tri_out_ref):
    i = pl.program_id(0)
    rows = lax.broadcasted_iota(jnp.int32, (CHUNK_CELLS, 1), 0) \
        + i * CHUNK_CELLS
    mask = rows < N_CELLS
    base = (jnp.dot(cc_ref[...], wcc_ref[...],
                    preferred_element_type=jnp.float32)
            + jnp.dot(vp_ref[...], wvp_ref[...],
                      preferred_element_type=jnp.float32)
            + c_ref[...])
    base = jnp.where(mask, base, 0.0)
    ffd = jnp.where(mask, ffd_ref[...], 0.0)
    bf_ref[...] = jnp.concatenate(
        [base, ffd, jnp.zeros((CHUNK_CELLS, 2), jnp.float32)], axis=1)
    tri_out_ref[...] = jnp.where(mask, tri_ref[...], 0)


def _prep(cc, vp6, ffd, tri, wcc, wvp, const):
    clamp = (N_CELLS - 1) // CHUNK_CELLS

    def im(i):
        return (jnp.minimum(i, clamp), 0)

    return pl.pallas_call(
        _prep_body,
        grid=(NUM_CHUNKS,),
        in_specs=[
            pl.BlockSpec((CHUNK_CELLS, 2), im),
            pl.BlockSpec((CHUNK_CELLS, 6), im),
            pl.BlockSpec((CHUNK_CELLS, 3), im),
            pl.BlockSpec((CHUNK_CELLS, 3), im),
            pl.BlockSpec((2, 3), lambda i: (0, 0)),
            pl.BlockSpec((6, 3), lambda i: (0, 0)),
            pl.BlockSpec((1, 3), lambda i: (0, 0)),
        ],
        out_specs=[
            pl.BlockSpec((CHUNK_CELLS, 8), lambda i: (i, 0)),
            pl.BlockSpec((CHUNK_CELLS, 3), lambda i: (i, 0)),
        ],
        out_shape=[
            jax.ShapeDtypeStruct((NC_PAD, 8), jnp.float32),
            jax.ShapeDtypeStruct((NC_PAD, 3), jnp.int32),
        ],
    )(cc, vp6, ffd, tri, wcc, wvp, const)


def _sc_body(q_hbm, tri_hbm, bf_hbm, zeros_hbm, out0_hbm, out1_hbm,
             idx_v, rows_v, bf_v, contrib_v, sem, ssem, accum_sh):
    core = lax.axis_index("c")
    sub = lax.axis_index("s")
    wid = core * NUM_SUBCORES + sub

    @pl.when(sub == 0)
    def _():
        pltpu.sync_copy(zeros_hbm, accum_sh)

    plsc.subcore_barrier()

    iota = lax.iota(jnp.int32, 16)
    three_iota = iota * 3
    lane_off = [jnp.full((16,), 3 * j + i, jnp.int32)
                for j in range(3) for i in range(3)]
    col_off = [jnp.full((16,), i, jnp.int32) for i in range(8)]

    def chunk_body(k, _):
        ch = wid + NUM_TILES * k

        @pl.when(ch <= LAST_REAL_CHUNK)
        def _():
            row0 = ch * CHUNK_STREAMS
            c0 = ch * CHUNK_CELLS

            pltpu.sync_copy(tri_hbm.at[pl.ds(row0, CHUNK_STREAMS)], idx_v)
            pltpu.sync_copy(bf_hbm.at[pl.ds(c0, CHUNK_CELLS)], bf_v)

            descs = []
            for s in range(CHUNK_STREAMS):
                descs.append(
                    pltpu.async_copy(q_hbm.at[idx_v.at[s]],
                                     rows_v.at[pl.ds(s * STREAM_B, STREAM_B)],
                                     sem))
            for d in descs:
                d.wait()

            for g in range(CHUNK_CELLS // 16):
                cells = iota + g * 16
                pair = [three_iota + (48 * g + k2) for k2 in range(3)]
                for i in range(3):
                    s0 = plsc.load_gather(rows_v, [pair[0], lane_off[0 + i]])
                    s1 = plsc.load_gather(rows_v, [pair[1], lane_off[3 + i]])
                    s2 = plsc.load_gather(rows_v, [pair[2], lane_off[6 + i]])
                    base_i = plsc.load_gather(bf_v, [cells, col_off[i]])
                    ffd_i = plsc.load_gather(bf_v, [cells, col_off[3 + i]])
                    contrib = ffd_i * (base_i + (s0 + s1) + s2)
                    plsc.store_scatter(contrib_v, [pair[i]], contrib)

            sdescs = []
            for s in range(CHUNK_STREAMS):
                sdescs.append(
                    pltpu.async_copy(contrib_v.at[pl.ds(s * STREAM_B,
                                                        STREAM_B)],
                                     accum_sh.at[idx_v.at[s]], ssem,
                                     add=True))
            for d in sdescs:
                d.wait()

        return ()

    lax.fori_loop(0, CHUNKS_PER_TILE, chunk_body, ())

    plsc.subcore_barrier()

    @pl.when(jnp.logical_and(sub == 0, core == 0))
    def _():
        pltpu.sync_copy(accum_sh, out0_hbm)

    @pl.when(jnp.logical_and(sub == 0, core == 1))
    def _():
        pltpu.sync_copy(accum_sh, out1_hbm)


def _sc_scatter(q, tri_rows, bf, zeros):
    mesh = plsc.VectorSubcoreMesh(core_axis_name="c", subcore_axis_name="s")
    kern = pl.kernel(
        _sc_body,
        out_type=(jax.ShapeDtypeStruct((N_NODES,), jnp.float32),
                  jax.ShapeDtypeStruct((N_NODES,), jnp.float32)),
        mesh=mesh,
        compiler_params=pltpu.CompilerParams(needs_layout_passes=False,
                                             use_tc_tiling_on_sc=False),
        scratch_types=[
            pltpu.VMEM((CHUNK_STREAMS, STREAM_B), jnp.int32),
            pltpu.VMEM((CHUNK_STREAMS * STREAM_B, 16), jnp.float32),
            pltpu.VMEM((CHUNK_CELLS, 8), jnp.float32),
            pltpu.VMEM((CHUNK_STREAMS * STREAM_B,), jnp.float32),
            pltpu.SemaphoreType.DMA,
            pltpu.SemaphoreType.DMA,
            pltpu.VMEM_SHARED((N_NODES,), jnp.float32),
        ],
    )
    return kern(q, tri_rows, bf, zeros)


def _combine_body(p0_ref, p1_ref, m_ref, o_ref):
    o_ref[...] = (p0_ref[...] + p1_ref[...]) * m_ref[...]


def _combine(p0, p1, inv_mass2d):
    return pl.pallas_call(
        _combine_body,
        out_shape=jax.ShapeDtypeStruct((1, N_NODES), jnp.float32),
    )(p0[None, :], p1[None, :], inv_mass2d)


@jax.jit
def kernel(u, t, triangulation, cell_centers, cell_local_vertex_pos,
           free_form_data, inv_mass, W, b):
    u2d = u[0]

    wv = W[9:].reshape(3, D, 3)
    wcat = jnp.transpose(wv, (1, 0, 2)).reshape(D, 9)
    wcat = jnp.pad(wcat, ((0, 0), (0, 7)))
    q = _compute_q(u2d, wcat)

    vp6 = cell_local_vertex_pos.reshape(N_CELLS, 6)
    const = (t[0, 0] * W[0] + b)[None, :]
    bf8, tri_flat = _prep(cell_centers, vp6, free_form_data, triangulation,
                          W[1:3], W[3:9], const)
    tri_rows = tri_flat.reshape(TRI_ROWS, STREAM_B)

    zeros = jnp.zeros((N_NODES,), jnp.float32)

    p0, p1 = _sc_scatter(q, tri_rows, bf8, zeros)

    return _combine(p0, p1, inv_mass[None, :])
